# Initial kernel scaffold; baseline (speedup 1.0000x reference)
import jax
import jax.numpy as jnp
from jax.experimental import pallas as pl


def kernel(pitch, envelope, attention, W):
    raise NotImplementedError("write your pallas kernel here")


# speedup vs baseline: 3.1072x; 3.1072x over previous
import jax
import jax.numpy as jnp
from jax.experimental import pallas as pl
from jax.experimental.pallas import tpu as pltpu

_INC = 512.0 / 16000.0
_NWT = 64
_WTLEN = 512


def _body(pitch_ref, pitch0_ref, env_ref, att_ref, vt_ref, out_ref, carry_ref):
    T = pitch_ref.shape[1]
    t = pl.program_id(1)

    @pl.when(t == 0)
    def _():
        carry_ref[...] = jnp.zeros_like(carry_ref)

    inc = pitch_ref[0] * _INC
    s = inc
    k = 1
    while k < T:
        s = s + jnp.concatenate(
            [jnp.zeros((k, 1), jnp.float32), s[: T - k]], axis=0
        )
        k *= 2
    carry = carry_ref[0:1, 0:1]
    s = s + carry
    carry_ref[0:1, 0:1] = s[T - 1 : T, :]

    index = s - pitch0_ref[0] * _INC
    index = jnp.remainder(index, float(_WTLEN))
    index = jnp.where(float(_WTLEN) - index < 1e-5, 0.0, index)
    fl = jnp.floor(index)
    alpha = index - fl
    fh = jnp.where(fl >= float(_WTLEN - 1), fl - float(_WTLEN - 1), fl + 1.0)

    il = fl.astype(jnp.int32)
    ih = fh.astype(jnp.int32)
    lane = jax.lax.broadcasted_iota(jnp.int32, (T, _WTLEN), 1)
    albc = jnp.broadcast_to(alpha, (T, _WTLEN))
    m = jnp.where(lane == jnp.broadcast_to(il, (T, _WTLEN)), 1.0 - albc, 0.0)
    m = m + jnp.where(lane == jnp.broadcast_to(ih, (T, _WTLEN)), albc, 0.0)

    mv = jnp.dot(m, vt_ref[...], preferred_element_type=jnp.float32)
    r = jnp.sum(mv * att_ref[0], axis=1, keepdims=True)
    out_ref[0] = r * env_ref[0]


def kernel(pitch, envelope, attention, W):
    B, L, _ = pitch.shape
    T = 3200 if L % 3200 == 0 else L
    NT = L // T
    vt = W.T
    out = pl.pallas_call(
        _body,
        grid=(B, NT),
        in_specs=[
            pl.BlockSpec((1, T, 1), lambda b, t: (b, t, 0)),
            pl.BlockSpec((1, T, 1), lambda b, t: (0, t, 0)),
            pl.BlockSpec((1, T, 1), lambda b, t: (b, t, 0)),
            pl.BlockSpec((1, T, _NWT), lambda b, t: (b, t, 0)),
            pl.BlockSpec((_WTLEN, _NWT), lambda b, t: (0, 0)),
        ],
        out_specs=pl.BlockSpec((1, T, 1), lambda b, t: (b, t, 0)),
        out_shape=jax.ShapeDtypeStruct((B, L, 1), jnp.float32),
        scratch_shapes=[pltpu.VMEM((8, 128), jnp.float32)],
    )(pitch, pitch, envelope, attention, vt)
    return out


# speedup vs baseline: 4.3401x; 1.3968x over previous
import jax
import jax.numpy as jnp
from jax.experimental import pallas as pl
from jax.experimental.pallas import tpu as pltpu

_INC = 512.0 / 16000.0
_NWT = 64
_WTLEN = 512


def _body(pitch_ref, pitch0_ref, env_ref, att_ref, w_ref, ones_ref, out_ref,
          carry_ref):
    T = pitch_ref.shape[-1]
    t = pl.program_id(1)

    @pl.when(t == 0)
    def _():
        carry_ref[...] = jnp.zeros_like(carry_ref)

    inc = pitch_ref[0, 0] * _INC
    s = inc
    k = 1
    while k < T:
        s = s + jnp.concatenate(
            [jnp.zeros((1, k), jnp.float32), s[:, : T - k]], axis=1
        )
        k *= 2
    carry = carry_ref[0:1, 0:1]
    s = s + carry
    carry_ref[0:1, 0:1] = s[:, T - 1 : T]

    index = s - pitch0_ref[0, 0] * _INC
    index = jnp.remainder(index, float(_WTLEN))
    index = jnp.where(float(_WTLEN) - index < 1e-5, 0.0, index)
    fl = jnp.floor(index)
    alpha = index - fl
    fh = jnp.where(fl >= float(_WTLEN - 1), fl - float(_WTLEN - 1), fl + 1.0)
    il = fl.astype(jnp.int32)
    ih = fh.astype(jnp.int32)

    row = jax.lax.broadcasted_iota(jnp.int32, (_WTLEN, T), 0)
    albc = jnp.broadcast_to(alpha, (_WTLEN, T))
    mt = jnp.where(row == jnp.broadcast_to(il, (_WTLEN, T)), 1.0 - albc, 0.0)
    mt = mt + jnp.where(row == jnp.broadcast_to(ih, (_WTLEN, T)), albc, 0.0)

    gt = jnp.dot(w_ref[...], mt, preferred_element_type=jnp.float32)
    g = gt.T
    prod = g * att_ref[0]
    r = jnp.dot(prod, ones_ref[...], preferred_element_type=jnp.float32)
    out_ref[0] = r * env_ref[0]


def kernel(pitch, envelope, attention, W):
    B, L, _ = pitch.shape
    T = 3200 if L % 3200 == 0 else L
    NT = L // T
    p2 = pitch.reshape(B, NT, 1, T)
    ones = jnp.ones((_NWT, 1), jnp.float32)
    out = pl.pallas_call(
        _body,
        grid=(B, NT),
        in_specs=[
            pl.BlockSpec((1, 1, 1, T), lambda b, t: (b, t, 0, 0)),
            pl.BlockSpec((1, 1, 1, T), lambda b, t: (0, t, 0, 0)),
            pl.BlockSpec((1, T, 1), lambda b, t: (b, t, 0)),
            pl.BlockSpec((1, T, _NWT), lambda b, t: (b, t, 0)),
            pl.BlockSpec((_NWT, _WTLEN), lambda b, t: (0, 0)),
            pl.BlockSpec((_NWT, 1), lambda b, t: (0, 0)),
        ],
        out_specs=pl.BlockSpec((1, T, 1), lambda b, t: (b, t, 0)),
        out_shape=jax.ShapeDtypeStruct((B, L, 1), jnp.float32),
        scratch_shapes=[pltpu.VMEM((8, 128), jnp.float32)],
    )(p2, p2, envelope, attention, W, ones)
    return out


# speedup vs baseline: 4.8390x; 1.1149x over previous
import jax
import jax.numpy as jnp
from jax.experimental import pallas as pl
from jax.experimental.pallas import tpu as pltpu

_INC = 512.0 / 16000.0
_NWT = 64
_WTLEN = 512


def _body(pitch_ref, pitch0_ref, env_ref, att_ref, w_ref, ones_ref, out_ref,
          carry_ref):
    T = pitch_ref.shape[-1]
    t = pl.program_id(1)

    @pl.when(t == 0)
    def _():
        carry_ref[...] = jnp.zeros_like(carry_ref)

    inc = pitch_ref[0, 0] * _INC
    s = inc
    k = 1
    while k < T:
        s = s + jnp.concatenate(
            [jnp.zeros((1, k), jnp.float32), s[:, : T - k]], axis=1
        )
        k *= 2
    carry = carry_ref[0:1, 0:1]
    s = s + carry
    carry_ref[0:1, 0:1] = s[:, T - 1 : T]

    index = s - pitch0_ref[0, 0] * _INC
    index = jnp.remainder(index, float(_WTLEN))
    index = jnp.where(float(_WTLEN) - index < 1e-5, 0.0, index)
    fl = jnp.floor(index)
    alpha = index - fl
    fh = jnp.where(fl >= float(_WTLEN - 1), fl - float(_WTLEN - 1), fl + 1.0)
    il = fl.astype(jnp.int32)
    ih = fh.astype(jnp.int32)

    row = jax.lax.broadcasted_iota(jnp.int32, (_WTLEN, T), 0)
    albc = jnp.broadcast_to(alpha, (_WTLEN, T))
    mt = jnp.where(row == jnp.broadcast_to(il, (_WTLEN, T)), 1.0 - albc, 0.0)
    mt = mt + jnp.where(row == jnp.broadcast_to(ih, (_WTLEN, T)), albc, 0.0)

    gt = jnp.dot(w_ref[...], mt, preferred_element_type=jnp.float32)
    g = gt.T
    prod = g * att_ref[0]
    r = jnp.dot(prod, ones_ref[...], preferred_element_type=jnp.float32)
    out_ref[0] = r * env_ref[0]


def kernel(pitch, envelope, attention, W):
    B, L, _ = pitch.shape
    T = 9600 if L % 9600 == 0 else L
    NT = L // T
    p2 = pitch.reshape(B, NT, 1, T)
    ones = jnp.ones((_NWT, 1), jnp.float32)
    out = pl.pallas_call(
        _body,
        grid=(B, NT),
        in_specs=[
            pl.BlockSpec((1, 1, 1, T), lambda b, t: (b, t, 0, 0)),
            pl.BlockSpec((1, 1, 1, T), lambda b, t: (0, t, 0, 0)),
            pl.BlockSpec((1, T, 1), lambda b, t: (b, t, 0)),
            pl.BlockSpec((1, T, _NWT), lambda b, t: (b, t, 0)),
            pl.BlockSpec((_NWT, _WTLEN), lambda b, t: (0, 0)),
            pl.BlockSpec((_NWT, 1), lambda b, t: (0, 0)),
        ],
        out_specs=pl.BlockSpec((1, T, 1), lambda b, t: (b, t, 0)),
        out_shape=jax.ShapeDtypeStruct((B, L, 1), jnp.float32),
        scratch_shapes=[pltpu.VMEM((8, 128), jnp.float32)],
    )(p2, p2, envelope, attention, W, ones)
    return out
